# baseline (speedup 1.0000x reference)
import functools
import math

import jax
import jax.numpy as jnp
from jax import lax
from jax.experimental import pallas as pl
from jax.experimental.pallas import tpu as pltpu
from jax.experimental.pallas import tpu_sc as plsc


def _pos_encoding(seq_len, dim):
    position = jnp.arange(0, seq_len, dtype=jnp.float32)[:, None]
    div_term = jnp.exp(
        jnp.arange(0, dim, 2, dtype=jnp.float32) * -(math.log(10000.0) / dim)
    )
    pe = jnp.zeros((seq_len, dim), dtype=jnp.float32)
    pe = pe.at[:, 0::2].set(jnp.sin(position * div_term))
    pe = pe.at[:, 1::2].set(jnp.cos(position * div_term))
    return pe


@functools.partial(jax.jit, static_argnums=(3, 4))
def _sc_embed(idx, pe, table, batch, seq):
    dim = table.shape[1]
    NC, NS = 2, 16
    NW = NC * NS
    L = 16
    BT = 2
    blk = BT * 128
    n_bt = batch // 128
    n_blocks = seq * (n_bt // BT)
    per_w = n_blocks // NW
    n_dt = dim // 8

    mesh = plsc.VectorSubcoreMesh(core_axis_name="c", subcore_axis_name="s")

    @functools.partial(
        pl.kernel,
        mesh=mesh,
        out_type=jax.ShapeDtypeStruct((seq, n_dt, n_bt, 8, 128), jnp.float32),
        scratch_types=[
            pltpu.VMEM((seq, dim), jnp.float32),
            pltpu.VMEM((dim, L), jnp.float32),
            pltpu.VMEM((blk,), jnp.int32),
            pltpu.VMEM((blk,), jnp.int32),
            pltpu.VMEM((blk, dim), jnp.float32),
            pltpu.VMEM((blk, dim), jnp.float32),
            pltpu.VMEM((n_dt, BT, 8, 128), jnp.float32),
            pltpu.VMEM((n_dt, BT, 8, 128), jnp.float32),
            pltpu.SemaphoreType.DMA,
            pltpu.SemaphoreType.DMA,
            pltpu.SemaphoreType.DMA,
            pltpu.SemaphoreType.DMA,
            pltpu.SemaphoreType.DMA,
            pltpu.SemaphoreType.DMA,
        ],
        compiler_params=pltpu.CompilerParams(
            use_tc_tiling_on_sc=False, needs_layout_passes=False),
    )
    def body(idx_hbm, pe_hbm, table_hbm, out_hbm,
             pe_v, pes_v, idx0, idx1, rows0, rows1, tiles0, tiles1,
             gs0, gs1, is0, is1, os0, os1):
        wid = lax.axis_index("s") * NC + lax.axis_index("c")
        first = wid * per_w
        pltpu.sync_copy(pe_hbm, pe_v)

        def block_pos(t):
            bid = first + t
            s = bid // (n_bt // BT)
            btp = lax.rem(bid, n_bt // BT)
            return s, btp

        def idx_offset(s, bc):
            return ((s // 8) * n_bt + bc) * 1024 + lax.rem(s, 8) * 128

        def issue_idx(t, idxb, isem):
            s, btp = block_pos(t)
            for h in range(BT):
                pltpu.async_copy(
                    idx_hbm.at[pl.ds(idx_offset(s, btp * BT + h), 128)],
                    idxb.at[pl.ds(h * 128, 128)], isem)

        def wait_idx(idxb, isem):
            for h in range(BT):
                pltpu.make_async_copy(
                    idx_hbm.at[pl.ds(0, 128)],
                    idxb.at[pl.ds(h * 128, 128)], isem).wait()

        def issue_gather(idxb, rowsb, gsem):
            for h in range(BT):
                pltpu.async_copy(
                    table_hbm.at[idxb.at[pl.ds(h * 128, 128)]],
                    rowsb.at[pl.ds(h * 128, 128)], gsem)

        bufs = ((idx0, rows0, tiles0, gs0, is0, os0),
                (idx1, rows1, tiles1, gs1, is1, os1))

        for b in range(2):
            idxb, rowsb, _, gsem, isem, _ = bufs[b]
            issue_idx(b, idxb, isem)
            wait_idx(idxb, isem)
            issue_gather(idxb, rowsb, gsem)

        iota = lax.iota(jnp.int32, L)

        def process(t, buf):
            idxb, rowsb, tilesb, gsem, isem, osem = buf
            s, btp = block_pos(t)
            srow = jnp.full((L,), s, jnp.int32)

            @plsc.parallel_loop(0, dim, 1, unroll=8)
            def _(d):
                pes_v[d, :] = plsc.load_gather(
                    pe_v, (srow, jnp.full((L,), d, jnp.int32)))

            pltpu.make_async_copy(table_hbm.at[idxb], rowsb, gsem).wait()
            @pl.when(t + 2 < per_w)
            def _():
                issue_idx(t + 2, idxb, isem)
            @pl.when(t >= 2)
            def _():
                for dt in range(n_dt):
                    pltpu.make_async_copy(
                        tilesb.at[dt],
                        out_hbm.at[0, dt, pl.ds(0, BT)], osem).wait()

            def dtile(dt, carry):
                for di in range(8):
                    pev = pes_v[dt * 8 + di, :]
                    col = jnp.full((L,), dt * 8 + di, jnp.int32)
                    for g in range(blk // L):
                        v = plsc.load_gather(rowsb, (iota + g * L, col))
                        tilesb[dt, g // 8, di, pl.ds((g % 8) * L, L)] = (
                            v + pev)
                return carry

            lax.fori_loop(0, n_dt, dtile, 0)

            for dt in range(n_dt):
                pltpu.async_copy(
                    tilesb.at[dt],
                    out_hbm.at[s, dt, pl.ds(btp * BT, BT)], osem)
            @pl.when(t + 2 < per_w)
            def _():
                wait_idx(idxb, isem)
                issue_gather(idxb, rowsb, gsem)

        def step(g, carry):
            process(2 * g, bufs[0])
            process(2 * g + 1, bufs[1])
            return carry

        lax.fori_loop(0, per_w // 2, step, 0)

        for b in range(2):
            _, _, tilesb, _, _, osem = bufs[b]
            for dt in range(n_dt):
                pltpu.make_async_copy(
                    tilesb.at[dt],
                    out_hbm.at[0, dt, pl.ds(0, BT)], osem).wait()

    return body(idx, pe, table)


def kernel(x, table):
    batch, seq = x.shape
    dim = table.shape[1]
    idx = (
        x.T.reshape(seq // 8, 8, batch // 128, 128)
        .transpose(0, 2, 1, 3)
        .reshape(-1)
        .astype(jnp.int32)
    )
    pe = _pos_encoding(seq, dim)
    out5 = _sc_embed(idx, pe, table, batch, seq)
    return out5.transpose(2, 4, 0, 1, 3).reshape(batch, seq, dim)


# speedup vs baseline: 1.5988x; 1.5988x over previous
import functools
import math

import jax
import jax.numpy as jnp
from jax import lax
from jax.experimental import pallas as pl
from jax.experimental.pallas import tpu as pltpu
from jax.experimental.pallas import tpu_sc as plsc


def _pos_encoding(seq_len, dim):
    position = jnp.arange(0, seq_len, dtype=jnp.float32)[:, None]
    div_term = jnp.exp(
        jnp.arange(0, dim, 2, dtype=jnp.float32) * -(math.log(10000.0) / dim)
    )
    pe = jnp.zeros((seq_len, dim), dtype=jnp.float32)
    pe = pe.at[:, 0::2].set(jnp.sin(position * div_term))
    pe = pe.at[:, 1::2].set(jnp.cos(position * div_term))
    return pe


@functools.partial(jax.jit, static_argnums=(3, 4))
def _sc_embed(idx, pe, table, batch, seq):
    n_rows = batch * seq
    dim = table.shape[1]
    NC, NS = 2, 16
    NW = NC * NS
    n_chunks = batch // NW
    s_a = 128
    s_b = seq - s_a

    mesh = plsc.VectorSubcoreMesh(core_axis_name="c", subcore_axis_name="s")

    @functools.partial(
        pl.kernel,
        mesh=mesh,
        out_type=jax.ShapeDtypeStruct((n_rows, dim), jnp.float32),
        scratch_types=[
            pltpu.VMEM((seq, dim), jnp.float32),
            pltpu.VMEM((seq,), jnp.int32),
            pltpu.VMEM((seq,), jnp.int32),
            pltpu.VMEM((seq, dim), jnp.float32),
            pltpu.VMEM((seq, dim), jnp.float32),
            pltpu.VMEM((seq, dim), jnp.float32),
            pltpu.VMEM((seq, dim), jnp.float32),
            pltpu.SemaphoreType.DMA,
            pltpu.SemaphoreType.DMA,
            pltpu.SemaphoreType.DMA,
            pltpu.SemaphoreType.DMA,
            pltpu.SemaphoreType.DMA,
            pltpu.SemaphoreType.DMA,
        ],
        compiler_params=pltpu.CompilerParams(use_tc_tiling_on_sc=False),
    )
    def body(idx_hbm, pe_hbm, table_hbm, out_hbm,
             pe_v, idx0, idx1, rows0, rows1, outs0, outs1,
             gs0, gs1, is0, is1, os0, os1):
        wid = lax.axis_index("s") * NC + lax.axis_index("c")
        first = wid * n_chunks
        pltpu.sync_copy(pe_hbm, pe_v)

        def issue_gather(t, idxb, rowsb, gsem):
            base = (first + t) * seq
            pltpu.async_copy(
                table_hbm.at[idxb.at[pl.ds(0, s_a)]],
                rowsb.at[pl.ds(0, s_a)], gsem)
            pltpu.async_copy(
                table_hbm.at[idxb.at[pl.ds(s_a, s_b)]],
                rowsb.at[pl.ds(s_a, s_b)], gsem)

        bufs = ((idx0, rows0, outs0, gs0, is0, os0),
                (idx1, rows1, outs1, gs1, is1, os1))

        for b in range(2):
            idxb, rowsb, _, gsem, _, _ = bufs[b]
            base = (first + b) * seq
            pltpu.sync_copy(idx_hbm.at[pl.ds(base, seq)], idxb)
            issue_gather(b, idxb, rowsb, gsem)

        def process(t, buf):
            idxb, rowsb, outb, gsem, isem, osem = buf
            pltpu.make_async_copy(
                table_hbm.at[idxb], rowsb, gsem).wait()
            @pl.when(t + 2 < n_chunks)
            def _():
                base2 = (first + t + 2) * seq
                pltpu.async_copy(idx_hbm.at[pl.ds(base2, seq)], idxb, isem)
            @pl.when(t >= 2)
            def _():
                pltpu.make_async_copy(
                    outb, out_hbm.at[pl.ds(0, seq)], osem).wait()

            @plsc.parallel_loop(0, seq, 1, unroll=8)
            def _(i):
                for c in range(dim // 16):
                    sl = pl.ds(c * 16, 16)
                    outb[i, sl] = rowsb[i, sl] + pe_v[i, sl]

            base = (first + t) * seq
            pltpu.async_copy(outb, out_hbm.at[pl.ds(base, seq)], osem)
            @pl.when(t + 2 < n_chunks)
            def _():
                pltpu.make_async_copy(
                    idx_hbm.at[pl.ds(0, seq)], idxb, isem).wait()
                issue_gather(t + 2, idxb, rowsb, gsem)

        def step(g, carry):
            process(2 * g, bufs[0])
            process(2 * g + 1, bufs[1])
            return carry

        lax.fori_loop(0, n_chunks // 2, step, 0)

        for b in range(2):
            _, _, outb, _, _, osem = bufs[b]
            pltpu.make_async_copy(outb, out_hbm.at[pl.ds(0, seq)], osem).wait()

    return body(idx, pe, table)


def kernel(x, table):
    batch, seq = x.shape
    dim = table.shape[1]
    idx = x.reshape(-1).astype(jnp.int32)
    pe = _pos_encoding(seq, dim)
    out = _sc_embed(idx, pe, table, batch, seq)
    return out.reshape(batch, seq, dim)
